# baseline (speedup 1.0000x reference)
import functools

import jax
import jax.numpy as jnp
from jax import lax
from jax.experimental import pallas as pl
from jax.experimental.pallas import tpu as pltpu
from jax.experimental.pallas import tpu_sc as plsc

D = 128
S = 10
C = 32
NW = 32
LANES = 16
CS = C * S


def _build_sc_kernel(n_pad: int):
    k_chunks = n_pad // (NW * C)
    mesh = plsc.VectorSubcoreMesh(core_axis_name="c", subcore_axis_name="s")

    @functools.partial(
        pl.kernel,
        mesh=mesh,
        out_type=jax.ShapeDtypeStruct((n_pad, D), jnp.float32),
        scratch_types=[
            pltpu.VMEM((k_chunks * CS,), jnp.int32),
            pltpu.VMEM((2, CS, D), jnp.float32),
            pltpu.VMEM((2, C, D), jnp.float32),
            pltpu.SemaphoreType.DMA,
            pltpu.SemaphoreType.DMA,
            pltpu.SemaphoreType.DMA,
            pltpu.SemaphoreType.DMA,
        ],
    )
    def sc_kernel(feat_hbm, idx_hbm, out_hbm, idx_v, rows_v, out_v,
                  semg0, semg1, semo0, semo1):
        wid = lax.axis_index("s") * 2 + lax.axis_index("c")
        semg = (semg0, semg1)
        semo = (semo0, semo1)

        pltpu.sync_copy(idx_hbm.at[wid], idx_v)

        def gather_copies(cj, b):
            base = pl.multiple_of(cj * CS, 64)
            return [
                pltpu.make_async_copy(
                    feat_hbm.at[idx_v.at[pl.ds(base + q, w)]],
                    rows_v.at[b].at[pl.ds(q, w)],
                    semg[b],
                )
                for q, w in ((0, 128), (128, 128), (256, 64))
            ]

        def fire_gathers(cj, b):
            for c in gather_copies(cj, b):
                c.start()

        def wait_gathers(cj, b):
            for c in gather_copies(cj, b):
                c.wait()

        def store_copy(cj, b):
            return pltpu.make_async_copy(
                out_v.at[b],
                out_hbm.at[pl.ds((wid * k_chunks + cj) * C, C)],
                semo[b],
            )

        def wait_store(cj, b):
            store_copy(cj, b).wait()

        def compute(cj, b):
            def node_body(i, c2):
                r = i * S
                for g in range(D // LANES):
                    col = pl.ds(g * LANES, LANES)
                    acc = rows_v[b, r, col]
                    for s in range(1, S):
                        acc = jnp.maximum(acc, rows_v[b, r + s, col])
                    out_v[b, i, col] = acc
                return c2

            lax.fori_loop(0, C, node_body, 0)
            store_copy(cj, b).start()

        def chunk_body(cj, carry):
            fire_gathers(cj, 0)
            wait_gathers(cj, 0)
            compute(cj, 0)
            wait_store(cj, 0)
            return carry

        lax.fori_loop(0, k_chunks, chunk_body, 0)

    return sc_kernel


def kernel(features, nodes, nbrs, num_sample):
    del nodes, num_sample
    n = features.shape[0]
    blk = NW * C * 2
    n_pad = ((n + blk - 1) // blk) * blk
    nbrs32 = jnp.pad(nbrs.astype(jnp.int32), ((0, n_pad - n), (0, 0)))
    idx_flat = nbrs32.reshape(NW, (n_pad // NW) * S)
    out = _build_sc_kernel(n_pad)(features, idx_flat)
    return out[:n]


# speedup vs baseline: 1.0005x; 1.0005x over previous
import functools

import jax
import jax.numpy as jnp
from jax import lax
from jax.experimental import pallas as pl
from jax.experimental.pallas import tpu as pltpu
from jax.experimental.pallas import tpu_sc as plsc

D = 128
S = 10
C = 32
NW = 32
LANES = 16
CS = C * S


def _build_sc_kernel(n_pad: int):
    k_chunks = n_pad // (NW * C)
    mesh = plsc.VectorSubcoreMesh(core_axis_name="c", subcore_axis_name="s")

    @functools.partial(
        pl.kernel,
        mesh=mesh,
        out_type=jax.ShapeDtypeStruct((n_pad, D), jnp.float32),
        scratch_types=[
            pltpu.VMEM((k_chunks * CS,), jnp.int32),
            pltpu.VMEM((2, CS, D), jnp.float32),
            pltpu.VMEM((2, C, D), jnp.float32),
            pltpu.SemaphoreType.DMA,
            pltpu.SemaphoreType.DMA,
            pltpu.SemaphoreType.DMA,
            pltpu.SemaphoreType.DMA,
        ],
    )
    def sc_kernel(feat_hbm, idx_hbm, out_hbm, idx_v, rows_v, out_v,
                  semg0, semg1, semo0, semo1):
        wid = lax.axis_index("s") * 2 + lax.axis_index("c")
        semg = (semg0, semg1)
        semo = (semo0, semo1)

        pltpu.sync_copy(idx_hbm.at[wid], idx_v)

        def gather_copies(cj, b):
            base = pl.multiple_of(cj * CS, 64)
            return [
                pltpu.make_async_copy(
                    feat_hbm.at[idx_v.at[pl.ds(base + s * C, C)]],
                    rows_v.at[b].at[pl.ds(s * C, C)],
                    semg[b],
                )
                for s in range(S)
            ]

        def fire_gathers(cj, b):
            for c in gather_copies(cj, b):
                c.start()

        def wait_gathers(cj, b):
            for c in gather_copies(cj, b):
                c.wait()

        def store_copy(cj, b):
            return pltpu.make_async_copy(
                out_v.at[b],
                out_hbm.at[pl.ds((wid * k_chunks + cj) * C, C)],
                semo[b],
            )

        def wait_store(cj, b):
            store_copy(cj, b).wait()

        def compute(cj, b):
            def node_body(i, c2):
                r = i * S
                for g in range(D // LANES):
                    col = pl.ds(g * LANES, LANES)
                    acc = rows_v[b, r, col]
                    for s in range(1, S):
                        acc = jnp.maximum(acc, rows_v[b, r + s, col])
                    out_v[b, i, col] = acc
                return c2

            lax.fori_loop(0, C, node_body, 0)
            store_copy(cj, b).start()

        def chunk_body(cj, carry):
            fire_gathers(cj, 0)
            wait_gathers(cj, 0)
            compute(cj, 0)
            wait_store(cj, 0)
            return carry

        lax.fori_loop(0, k_chunks, chunk_body, 0)

    return sc_kernel


def kernel(features, nodes, nbrs, num_sample):
    del nodes, num_sample
    n = features.shape[0]
    blk = NW * C * 2
    n_pad = ((n + blk - 1) // blk) * blk
    nbrs32 = jnp.pad(nbrs.astype(jnp.int32), ((0, n_pad - n), (0, 0)))
    idx_flat = nbrs32.reshape(NW, (n_pad // NW) * S)
    out = _build_sc_kernel(n_pad)(features, idx_flat)
    return out[:n]


# speedup vs baseline: 1.1377x; 1.1371x over previous
import functools

import jax
import jax.numpy as jnp
from jax import lax
from jax.experimental import pallas as pl
from jax.experimental.pallas import tpu as pltpu
from jax.experimental.pallas import tpu_sc as plsc

D = 128
S = 10
C = 32
NW = 32
LANES = 16


def _build_sc_kernel(n_pad: int):
    k_chunks = n_pad // (NW * C)
    mesh = plsc.VectorSubcoreMesh(core_axis_name="c", subcore_axis_name="s")

    @functools.partial(
        pl.kernel,
        mesh=mesh,
        out_type=jax.ShapeDtypeStruct((n_pad, D), jnp.float32),
        scratch_types=[
            pltpu.VMEM((S, C), jnp.int32),
            pltpu.VMEM((S, C), jnp.int32),
            pltpu.VMEM((S * C, D), jnp.float32),
            pltpu.VMEM((S * C, D), jnp.float32),
            pltpu.VMEM((C, D), jnp.float32),
            pltpu.VMEM((C, D), jnp.float32),
            pltpu.SemaphoreType.DMA,
            pltpu.SemaphoreType.DMA,
            pltpu.SemaphoreType.DMA,
            pltpu.SemaphoreType.DMA,
            pltpu.SemaphoreType.DMA,
            pltpu.SemaphoreType.DMA,
        ],
    )
    def sc_kernel(feat_hbm, idx_hbm, out_hbm,
                  idx0, idx1, rows0, rows1, outv0, outv1,
                  semi0, semi1, semg0, semg1, semo0, semo1):
        wid = lax.axis_index("s") * 2 + lax.axis_index("c")
        idxv = (idx0, idx1)
        rows = (rows0, rows1)
        outv = (outv0, outv1)
        semi = (semi0, semi1)
        semg = (semg0, semg1)
        semo = (semo0, semo1)

        def idx_copy(cj, b):
            return pltpu.make_async_copy(
                idx_hbm.at[wid * k_chunks + cj], idxv[b], semi[b]
            )

        def gather_copies(b):
            return [
                pltpu.make_async_copy(
                    feat_hbm.at[idxv[b].at[s]],
                    rows[b].at[pl.ds(s * C, C)],
                    semg[b],
                )
                for s in range(S)
            ]

        def fire_gathers(b):
            for cp in gather_copies(b):
                cp.start()

        def wait_gathers(b):
            for cp in gather_copies(b):
                cp.wait()

        def store_copy(cj, b):
            return pltpu.make_async_copy(
                outv[b],
                out_hbm.at[pl.ds((wid * k_chunks + cj) * C, C)],
                semo[b],
            )

        def wait_store(cj, b):
            store_copy(cj, b).wait()

        def compute(cj, b):
            def node_body(i, c2):
                for g in range(D // LANES):
                    col = pl.ds(g * LANES, LANES)
                    acc = rows[b][i, col]
                    for s in range(1, S):
                        acc = jnp.maximum(acc, rows[b][s * C + i, col])
                    outv[b][i, col] = acc
                return c2

            lax.fori_loop(0, C, node_body, 0)
            store_copy(cj, b).start()

        cp = idx_copy(0, 0)
        cp.start()
        cp.wait()
        fire_gathers(0)
        idx_copy(1, 1).start()

        def pair_body(p, carry):
            for b in range(2):
                cj = 2 * p + b
                nb = 1 - b
                wait_gathers(b)

                @pl.when(cj + 1 < k_chunks)
                def _():
                    idx_copy(cj + 1, nb).wait()
                    fire_gathers(nb)

                @pl.when(cj >= 2)
                def _():
                    wait_store(cj - 2, b)

                compute(cj, b)

                @pl.when(cj + 2 < k_chunks)
                def _():
                    idx_copy(cj + 2, b).start()

            return carry

        lax.fori_loop(0, k_chunks // 2, pair_body, 0)
        wait_store(k_chunks - 2, 0)
        wait_store(k_chunks - 1, 1)

    return sc_kernel


def kernel(features, nodes, nbrs, num_sample):
    del nodes, num_sample
    n = features.shape[0]
    blk = NW * C * 2
    n_pad = ((n + blk - 1) // blk) * blk
    nbrs32 = jnp.pad(nbrs.astype(jnp.int32), ((0, n_pad - n), (0, 0)))
    idx_chunks = nbrs32.reshape(n_pad // C, C, S).transpose(0, 2, 1)
    out = _build_sc_kernel(n_pad)(features, idx_chunks)
    return out[:n]


# speedup vs baseline: 2.8308x; 2.4881x over previous
import functools

import jax
import jax.numpy as jnp
from jax import lax
from jax.experimental import pallas as pl
from jax.experimental.pallas import tpu as pltpu
from jax.experimental.pallas import tpu_sc as plsc

D = 128
S = 10
C = 32
NW = 32
LANES = 16


def _build_sc_kernel(n_pad: int):
    chunks_per_w = n_pad // (NW * C)
    mesh = plsc.VectorSubcoreMesh(core_axis_name="c", subcore_axis_name="s")

    @functools.partial(
        pl.kernel,
        mesh=mesh,
        out_type=jax.ShapeDtypeStruct((n_pad, D), jnp.float32),
        scratch_types=[
            pltpu.VMEM((S, C), jnp.int32),
            pltpu.VMEM((S * C, D), jnp.float32),
            pltpu.VMEM((C, D), jnp.float32),
            pltpu.SemaphoreType.DMA,
        ],
    )
    def sc_kernel(feat_hbm, idx_hbm, out_hbm, idx_v, rows_v, out_v, sem):
        wid = lax.axis_index("s") * 2 + lax.axis_index("c")

        def chunk_body(j, carry):
            chunk = wid * chunks_per_w + j
            pltpu.sync_copy(idx_hbm.at[chunk], idx_v)
            handles = []
            for s in range(S):
                handles.append(
                    pltpu.async_copy(
                        feat_hbm.at[idx_v.at[s]], rows_v.at[pl.ds(s * C, C)], sem
                    )
                )
            for h in handles:
                h.wait()

            def node_body(i, c2):
                for g in range(D // LANES):
                    col = pl.ds(g * LANES, LANES)
                    acc = rows_v[i, col]
                    for s in range(1, S):
                        acc = jnp.maximum(acc, rows_v[s * C + i, col])
                    out_v[i, col] = acc
                return c2

            lax.fori_loop(0, C, node_body, 0)
            pltpu.sync_copy(out_v, out_hbm.at[pl.ds(chunk * C, C)])
            return carry

        lax.fori_loop(0, chunks_per_w, chunk_body, 0)

    return sc_kernel


def kernel(features, nodes, nbrs, num_sample):
    del nodes, num_sample
    n = features.shape[0]
    n_pad = ((n + NW * C - 1) // (NW * C)) * (NW * C)
    nbrs32 = jnp.pad(nbrs.astype(jnp.int32), ((0, n_pad - n), (0, 0)))
    idx_chunks = nbrs32.reshape(n_pad // C, C, S).transpose(0, 2, 1)
    out = _build_sc_kernel(n_pad)(features, idx_chunks)
    return out[:n]
